# baseline (speedup 1.0000x reference)
import jax
import jax.numpy as jnp
from jax import lax
from jax.experimental import pallas as pl
from jax.experimental.pallas import tpu as pltpu

N = 100
NP = 128
IN_DIM = 512
HID = 512
HB = HID // 2
H = 8
D = HID * H
FF = HID * 4
OUT_DIM = 256
K = 10
NEG = -1e30
F32 = jnp.float32

_CONTRACT_11 = (((1,), (1,)), ((), ()))
_CONTRACT_10 = (((1,), (0,)), ((), ()))


def _dot(a, b, dims):
    return lax.dot_general(a, b, dims, preferred_element_type=F32)


def _ln(xa):
    mu = jnp.mean(xa, axis=1, keepdims=True)
    var = jnp.mean((xa - mu) ** 2, axis=1, keepdims=True)
    return (xa - mu) / jnp.sqrt(var + 1e-5)


def _ab_maps(offset, nblocks, row_major):
    def a_map(j):
        jc = jnp.clip(j - offset, 0, nblocks - 1)
        b = jnp.minimum(2 * ((jc + 1) // 2), nblocks - 2)
        return (b, 0) if row_major else (0, b)

    def b_map(j):
        jc = jnp.clip(j - offset, 0, nblocks - 1)
        b = 2 * (jc // 2) + 1
        return (b, 0) if row_major else (0, b)

    return a_map, b_map


def _graph_setup(x, c_ref, stats_ref, cs):
    sims = _dot(x, x, _CONTRACT_11)
    col = lax.broadcasted_iota(jnp.int32, (NP, NP), 1)
    row = lax.broadcasted_iota(jnp.int32, (NP, NP), 0)
    valid_col = col < N
    valid_row = row < N

    selected = jnp.zeros((NP, NP), jnp.bool_)
    t_mat = jnp.zeros((NP, NP), F32)
    for t in range(K + 1):
        masked = jnp.where(valid_col & (~selected), sims, NEG)
        rowmax = jnp.max(masked, axis=1, keepdims=True)
        cand = jnp.where(masked == rowmax, col, NP)
        first = jnp.min(cand, axis=1, keepdims=True)
        newsel = col == first
        selected = selected | newsel
        if t > 0:
            t_mat = t_mat + newsel.astype(F32)
    t_mat = jnp.where(valid_row, t_mat, 0.0)
    cmat = t_mat.T + jnp.where((row == col) & valid_row, 1.0, 0.0)
    c_ref[...] = cmat
    cs[...] = cmat

    centrality = jnp.sum(sims, axis=1, keepdims=True)
    validr1 = lax.broadcasted_iota(jnp.int32, (NP, 1), 0) < N
    cmean = jnp.sum(centrality) / N
    cvar = jnp.sum(jnp.where(validr1, (centrality - cmean) ** 2, 0.0)) / (N - 1)
    cstd = jnp.sqrt(cvar)
    smean = jnp.sum(sims) / (N * N)
    degree = jnp.sum((sims > 0.5).astype(F32), axis=1, keepdims=True)
    s2 = _dot(sims, sims, _CONTRACT_10)
    tri = jnp.sum(s2 * sims, axis=1, keepdims=True)
    clus = tri / (degree * (degree - 1.0) + 1e-8)
    clustering = jnp.sum(jnp.where(validr1, clus, 0.0)) / N

    lane = lax.broadcasted_iota(jnp.int32, (8, 128), 1)
    rw = lax.broadcasted_iota(jnp.int32, (8, 128), 0)
    stats_ref[...] = jnp.where((rw == 0) & (lane == 0), cmean,
                     jnp.where((rw == 0) & (lane == 1), clustering,
                     jnp.where((rw == 0) & (lane == 2), smean,
                     jnp.where((rw == 0) & (lane == 3), cstd, 0.0))))


def _gat_attend(xp, asrc_ref, adst_ref, cmat, h):
    asr = asrc_ref[pl.ds(h, 1), :]
    adr = adst_ref[pl.ds(h, 1), :]
    a_s_row = _dot(asr, xp, _CONTRACT_11)
    a_d_col = _dot(xp, adr, _CONTRACT_11)
    e = a_s_row + a_d_col
    e = jnp.where(e >= 0, e, 0.2 * e)
    mask = cmat > 0.0
    em = jnp.where(mask, e, NEG)
    m = jnp.max(em, axis=1, keepdims=True)
    m = jnp.where(m > 0.5 * NEG, m, 0.0)
    ex = jnp.where(mask, jnp.exp(e - m), 0.0)
    z = jnp.sum(cmat * ex, axis=1, keepdims=True)
    alpha = ex / (z + 1e-16)
    out = _dot(cmat * alpha, xp, _CONTRACT_10)
    out = jnp.where(out > 0, out, jnp.exp(out) - 1.0)
    validr = lax.broadcasted_iota(jnp.int32, (NP, HID), 0) < N
    return jnp.where(validr, out, 0.0), alpha


def _k1_body(x_ref, pe_ref, w0_ref, w1a_ref, w1b_ref, w2a_ref, w2b_ref,
             as0_ref, ad0_ref, as1_ref, ad1_ref, as2_ref, ad2_ref,
             c_ref, g3_ref, v1_ref, v2_ref, v3_ref, stats_ref,
             g0s, gas, gbs, cs, vacc, xps):
    j = pl.program_id(0)

    @pl.when(j == 0)
    def _():
        x = x_ref[...]
        _graph_setup(x, c_ref, stats_ref, cs)
        validg = lax.broadcasted_iota(jnp.int32, (NP, IN_DIM), 0) < N
        g0s[...] = jnp.where(validg, x + pe_ref[...], 0.0)

    def head_epilogue(h, alpha, v_ref):
        @pl.when(h == 0)
        def _():
            vacc[...] = alpha * (1.0 / H)

        @pl.when(h > 0)
        def _():
            vacc[...] += alpha * (1.0 / H)

        @pl.when(h == H - 1)
        def _():
            v_ref[...] = vacc[...]

    @pl.when((j >= 1) & (j <= 8))
    def _():
        h = j - 1
        xp = _dot(g0s[...], w0_ref[...], _CONTRACT_10)
        out, alpha = _gat_attend(xp, as0_ref, ad0_ref, cs[...], h)
        gas[:, pl.ds(h * HID, HID)] = out
        head_epilogue(h, alpha, v1_ref)

    def half_layer(first_step, gin_ref, wa_ref, wb_ref, asr, adr,
                   out_write, v_ref):
        jc = j - first_step
        h = jc // 2

        @pl.when(jc % 2 == 0)
        def _():
            xps[:, :HB] = _dot(gin_ref[...], wa_ref[...], _CONTRACT_10)

        @pl.when(jc % 2 == 1)
        def _():
            xps[:, HB:] = _dot(gin_ref[...], wb_ref[...], _CONTRACT_10)
            out, alpha = _gat_attend(xps[...], asr, adr, cs[...], h)
            out_write(h, out)
            head_epilogue(h, alpha, v_ref)

    @pl.when((j >= 9) & (j <= 24))
    def _():
        half_layer(9, gas, w1a_ref, w1b_ref, as1_ref, ad1_ref,
                   lambda h, o: gbs.__setitem__(
                       (slice(None), pl.ds(h * HID, HID)), o), v2_ref)

    @pl.when(j >= 25)
    def _():
        half_layer(25, gbs, w2a_ref, w2b_ref, as2_ref, ad2_ref,
                   lambda h, o: g3_ref.__setitem__(
                       (slice(None), pl.ds(h * HID, HID)), o), v3_ref)


def _k1(xp, pep, gat):
    const2 = lambda _: (0, 0)
    w1a, w1b = _ab_maps(9, 2 * H, row_major=False)
    w2a, w2b = _ab_maps(25, 2 * H, row_major=False)
    return pl.pallas_call(
        _k1_body,
        grid=(1 + H + 4 * H,),
        in_specs=[
            pl.BlockSpec((NP, IN_DIM), const2),
            pl.BlockSpec((NP, IN_DIM), const2),
            pl.BlockSpec((IN_DIM, HID), lambda j: (0, jnp.clip(j - 1, 0, H - 1))),
            pl.BlockSpec((D, HB), w1a),
            pl.BlockSpec((D, HB), w1b),
            pl.BlockSpec((D, HB), w2a),
            pl.BlockSpec((D, HB), w2b),
            pl.BlockSpec((H, HID), const2), pl.BlockSpec((H, HID), const2),
            pl.BlockSpec((H, HID), const2), pl.BlockSpec((H, HID), const2),
            pl.BlockSpec((H, HID), const2), pl.BlockSpec((H, HID), const2),
        ],
        out_specs=(
            pl.BlockSpec((NP, NP), const2),
            pl.BlockSpec((NP, D), const2),
            pl.BlockSpec((NP, NP), const2),
            pl.BlockSpec((NP, NP), const2),
            pl.BlockSpec((NP, NP), const2),
            pl.BlockSpec((8, 128), const2),
        ),
        out_shape=(
            jax.ShapeDtypeStruct((NP, NP), F32),
            jax.ShapeDtypeStruct((NP, D), F32),
            jax.ShapeDtypeStruct((NP, NP), F32),
            jax.ShapeDtypeStruct((NP, NP), F32),
            jax.ShapeDtypeStruct((NP, NP), F32),
            jax.ShapeDtypeStruct((8, 128), F32),
        ),
        scratch_shapes=[
            pltpu.VMEM((NP, IN_DIM), F32),
            pltpu.VMEM((NP, D), F32),
            pltpu.VMEM((NP, D), F32),
            pltpu.VMEM((NP, NP), F32),
            pltpu.VMEM((NP, NP), F32),
            pltpu.VMEM((NP, HID), F32),
        ],
    )(xp, pep, gat[0]['W'], gat[1]['W'], gat[1]['W'], gat[2]['W'],
      gat[2]['W'], gat[0]['a_src'], gat[0]['a_dst'], gat[1]['a_src'],
      gat[1]['a_dst'], gat[2]['a_src'], gat[2]['a_dst'])


_QKV_STEPS = 6 * H
_ATT0 = _QKV_STEPS - 2 * H + 2
_PROJ0 = _QKV_STEPS + 1


def _k2_body(g_ref, wia_ref, wib_ref, woa_ref, wob_ref, x1_ref,
             qkvs, asc, accs):
    j = pl.program_id(0)

    @pl.when((j < _QKV_STEPS) & (j % 2 == 0))
    def _():
        qkvs[:, pl.ds(j * HB, HB)] = _dot(g_ref[...], wia_ref[...],
                                          _CONTRACT_11)

    @pl.when((j < _QKV_STEPS) & (j % 2 == 1))
    def _():
        qkvs[:, pl.ds(j * HB, HB)] = _dot(g_ref[...], wib_ref[...],
                                          _CONTRACT_11)

    @pl.when((j >= _ATT0) & (j <= _QKV_STEPS) & ((j - _ATT0) % 2 == 0))
    def _():
        h = (j - _ATT0) // 2
        qh = qkvs[:, pl.ds(h * HID, HID)]
        kh = qkvs[:, pl.ds(D + h * HID, HID)]
        vh = qkvs[:, pl.ds(2 * D + h * HID, HID)]
        logits = _dot(qh, kh, _CONTRACT_11) * (1.0 / jnp.sqrt(HID * 1.0))
        colmask = lax.broadcasted_iota(jnp.int32, (NP, NP), 1) < N
        logits = jnp.where(colmask, logits, NEG)
        m = jnp.max(logits, axis=1, keepdims=True)
        e = jnp.exp(logits - m)
        e = jnp.where(colmask, e, 0.0)
        att = e / jnp.sum(e, axis=1, keepdims=True)
        asc[:, pl.ds(h * HID, HID)] = _dot(att, vh, _CONTRACT_10)

    @pl.when(j >= _PROJ0)
    def _():
        jj = j - _PROJ0

        @pl.when(jj % 2 == 0)
        def _():
            accs[:, pl.ds(jj * HB, HB)] = _dot(asc[...], woa_ref[...],
                                               _CONTRACT_11)

        @pl.when(jj % 2 == 1)
        def _():
            accs[:, pl.ds(jj * HB, HB)] = _dot(asc[...], wob_ref[...],
                                               _CONTRACT_11)

        @pl.when(jj == 2 * H - 1)
        def _():
            x1_ref[...] = _ln(g_ref[...] + accs[...])


def _k2(g3, w_in, w_out):
    const2 = lambda _: (0, 0)
    ia, ib = _ab_maps(0, 6 * H, row_major=True)
    oa, ob = _ab_maps(_PROJ0, 2 * H, row_major=True)
    return pl.pallas_call(
        _k2_body,
        grid=(_PROJ0 + 2 * H,),
        in_specs=[
            pl.BlockSpec((NP, D), const2),
            pl.BlockSpec((HB, D), ia),
            pl.BlockSpec((HB, D), ib),
            pl.BlockSpec((HB, D), oa),
            pl.BlockSpec((HB, D), ob),
        ],
        out_specs=pl.BlockSpec((NP, D), const2),
        out_shape=jax.ShapeDtypeStruct((NP, D), F32),
        scratch_shapes=[
            pltpu.VMEM((NP, 3 * D), F32),
            pltpu.VMEM((NP, D), F32),
            pltpu.VMEM((NP, D), F32),
        ],
    )(g3, w_in, w_in, w_out, w_out)


def _edge_entropy(v, cmat, mask):
    vm = jnp.where(mask, v, NEG)
    mx = jnp.max(vm)
    e = jnp.where(mask, jnp.exp(v - mx), 0.0)
    s = jnp.sum(cmat * e)
    pr = e / s
    term = jnp.where(mask, pr * jnp.log(pr + 1e-8), 0.0)
    return -jnp.sum(cmat * term)


_F1_STEPS = 2 * FF // HID
_F2_STEPS = 2 * H


def _k3_body(x1_ref, w1a_ref, w1b_ref, w2a_ref, w2b_ref, wo_ref,
             c_ref, v1_ref, v2_ref, v3_ref, out_ref, st_ref, fs, accs):
    j = pl.program_id(0)

    @pl.when((j < _F1_STEPS) & (j % 2 == 0))
    def _():
        r = _dot(x1_ref[...], w1a_ref[...], _CONTRACT_11)
        fs[:, pl.ds(j * HB, HB)] = jnp.maximum(r, 0.0)

    @pl.when((j < _F1_STEPS) & (j % 2 == 1))
    def _():
        r = _dot(x1_ref[...], w1b_ref[...], _CONTRACT_11)
        fs[:, pl.ds(j * HB, HB)] = jnp.maximum(r, 0.0)

    @pl.when(j >= _F1_STEPS)
    def _():
        jj = j - _F1_STEPS

        @pl.when(jj % 2 == 0)
        def _():
            accs[:, pl.ds(jj * HB, HB)] = _dot(fs[...], w2a_ref[...],
                                               _CONTRACT_11)

        @pl.when(jj % 2 == 1)
        def _():
            accs[:, pl.ds(jj * HB, HB)] = _dot(fs[...], w2b_ref[...],
                                               _CONTRACT_11)

        @pl.when(jj == _F2_STEPS - 1)
        def _():
            t = _ln(x1_ref[...] + accs[...])
            validr = lax.broadcasted_iota(jnp.int32, (NP, D), 0) < N
            tmean = jnp.sum(jnp.where(validr, t, 0.0), axis=0,
                            keepdims=True) / N
            out = _dot(tmean, wo_ref[...], _CONTRACT_11)
            out_ref[...] = jnp.broadcast_to(out, (8, OUT_DIM))

            cmat = c_ref[...]
            mask = cmat > 0.0
            ent = (_edge_entropy(v1_ref[...], cmat, mask)
                   + _edge_entropy(v2_ref[...], cmat, mask)
                   + _edge_entropy(v3_ref[...], cmat, mask)) / 3.0
            rw = lax.broadcasted_iota(jnp.int32, (8, 128), 0)
            lane = lax.broadcasted_iota(jnp.int32, (8, 128), 1)
            st_ref[...] = jnp.where((rw == 0) & (lane == 0), ent, 0.0)


def _k3(x1, w1, w2, wo, cmat, v1, v2, v3):
    const2 = lambda _: (0, 0)
    f1a, f1b = _ab_maps(0, _F1_STEPS, row_major=True)
    f2a, f2b = _ab_maps(_F1_STEPS, _F2_STEPS, row_major=True)
    return pl.pallas_call(
        _k3_body,
        grid=(_F1_STEPS + _F2_STEPS,),
        in_specs=[
            pl.BlockSpec((NP, D), const2),
            pl.BlockSpec((HB, D), f1a),
            pl.BlockSpec((HB, D), f1b),
            pl.BlockSpec((HB, FF), f2a),
            pl.BlockSpec((HB, FF), f2b),
            pl.BlockSpec((OUT_DIM, D), const2),
            pl.BlockSpec((NP, NP), const2),
            pl.BlockSpec((NP, NP), const2),
            pl.BlockSpec((NP, NP), const2),
            pl.BlockSpec((NP, NP), const2),
        ],
        out_specs=(
            pl.BlockSpec((8, OUT_DIM), const2),
            pl.BlockSpec((8, 128), const2),
        ),
        out_shape=(
            jax.ShapeDtypeStruct((8, OUT_DIM), F32),
            jax.ShapeDtypeStruct((8, 128), F32),
        ),
        scratch_shapes=[
            pltpu.VMEM((NP, FF), F32),
            pltpu.VMEM((NP, D), F32),
        ],
    )(x1, w1, w1, w2, w2, wo, cmat, v1, v2, v3)


def kernel(x, params):
    xp = jnp.pad(x, ((0, NP - N), (0, 0)))
    pep = jnp.pad(params['topo_pe'][:N, :IN_DIM], ((0, NP - N), (0, 0)))

    cmat, g3, v1, v2, v3, stats = _k1(xp, pep, params['gat'])
    x1 = _k2(g3, params['in_proj_w'], params['out_proj_w'])
    outr, st2 = _k3(x1, params['ff1_w'], params['ff2_w'], params['outp_w'],
                    cmat, v1, v2, v3)

    out = outr[0]
    return (out, stats[0, 0], stats[0, 1], st2[0, 0], stats[0, 2], stats[0, 3])


# speedup vs baseline: 1.2339x; 1.2339x over previous
import jax
import jax.numpy as jnp
from jax import lax
from jax.experimental import pallas as pl
from jax.experimental.pallas import tpu as pltpu

N = 100
NP = 128
IN_DIM = 512
HID = 512
H = 8
D = HID * H
FF = HID * 4
OUT_DIM = 256
K = 10
NEG = -1e30
F32 = jnp.float32

_CONTRACT_11 = (((1,), (1,)), ((), ()))
_CONTRACT_10 = (((1,), (0,)), ((), ()))


def _dot(a, b, dims):
    return lax.dot_general(a, b, dims, preferred_element_type=F32)


def _ln(xa):
    mu = jnp.mean(xa, axis=1, keepdims=True)
    var = jnp.mean((xa - mu) ** 2, axis=1, keepdims=True)
    return (xa - mu) / jnp.sqrt(var + 1e-5)


def _graph_setup(x, c_ref, stats_ref, cs):
    sims = _dot(x, x, _CONTRACT_11)
    col = lax.broadcasted_iota(jnp.int32, (NP, NP), 1)
    row = lax.broadcasted_iota(jnp.int32, (NP, NP), 0)
    valid_col = col < N
    valid_row = row < N

    selected = jnp.zeros((NP, NP), jnp.bool_)
    t_mat = jnp.zeros((NP, NP), F32)
    for t in range(K + 1):
        masked = jnp.where(valid_col & (~selected), sims, NEG)
        rowmax = jnp.max(masked, axis=1, keepdims=True)
        cand = jnp.where(masked == rowmax, col, NP)
        first = jnp.min(cand, axis=1, keepdims=True)
        newsel = col == first
        selected = selected | newsel
        if t > 0:
            t_mat = t_mat + newsel.astype(F32)
    t_mat = jnp.where(valid_row, t_mat, 0.0)
    cmat = t_mat.T + jnp.where((row == col) & valid_row, 1.0, 0.0)
    c_ref[...] = cmat
    cs[...] = cmat

    centrality = jnp.sum(sims, axis=1, keepdims=True)
    validr1 = lax.broadcasted_iota(jnp.int32, (NP, 1), 0) < N
    cmean = jnp.sum(centrality) / N
    cvar = jnp.sum(jnp.where(validr1, (centrality - cmean) ** 2, 0.0)) / (N - 1)
    cstd = jnp.sqrt(cvar)
    smean = jnp.sum(sims) / (N * N)
    degree = jnp.sum((sims > 0.5).astype(F32), axis=1, keepdims=True)
    s2 = _dot(sims, sims, _CONTRACT_10)
    tri = jnp.sum(s2 * sims, axis=1, keepdims=True)
    clus = tri / (degree * (degree - 1.0) + 1e-8)
    clustering = jnp.sum(jnp.where(validr1, clus, 0.0)) / N

    lane = lax.broadcasted_iota(jnp.int32, (8, 128), 1)
    rw = lax.broadcasted_iota(jnp.int32, (8, 128), 0)
    stats_ref[...] = jnp.where((rw == 0) & (lane == 0), cmean,
                     jnp.where((rw == 0) & (lane == 1), clustering,
                     jnp.where((rw == 0) & (lane == 2), smean,
                     jnp.where((rw == 0) & (lane == 3), cstd, 0.0))))


def _gat_step(g, w, asrc_ref, adst_ref, cmat, h):
    xp = _dot(g, w, _CONTRACT_10)
    asr = asrc_ref[pl.ds(h, 1), :]
    adr = adst_ref[pl.ds(h, 1), :]
    a_s_row = _dot(asr, xp, _CONTRACT_11)
    a_d_col = _dot(xp, adr, _CONTRACT_11)
    e = a_s_row + a_d_col
    e = jnp.where(e >= 0, e, 0.2 * e)
    mask = cmat > 0.0
    em = jnp.where(mask, e, NEG)
    m = jnp.max(em, axis=1, keepdims=True)
    m = jnp.where(m > 0.5 * NEG, m, 0.0)
    ex = jnp.where(mask, jnp.exp(e - m), 0.0)
    z = jnp.sum(cmat * ex, axis=1, keepdims=True)
    alpha = ex / (z + 1e-16)
    out = _dot(cmat * alpha, xp, _CONTRACT_10)
    out = jnp.where(out > 0, out, jnp.exp(out) - 1.0)
    validr = lax.broadcasted_iota(jnp.int32, (NP, HID), 0) < N
    return jnp.where(validr, out, 0.0), alpha


def _k1_body(x_ref, pe_ref, w0_ref, w1_ref, w2_ref,
             as0_ref, ad0_ref, as1_ref, ad1_ref, as2_ref, ad2_ref,
             c_ref, g3_ref, v1_ref, v2_ref, v3_ref, stats_ref,
             g0s, gas, gbs, cs, vacc):
    j = pl.program_id(0)

    @pl.when(j == 0)
    def _():
        x = x_ref[...]
        _graph_setup(x, c_ref, stats_ref, cs)
        validg = lax.broadcasted_iota(jnp.int32, (NP, IN_DIM), 0) < N
        g0s[...] = jnp.where(validg, x + pe_ref[...], 0.0)

    def layer(first_step, gin, w_ref, asr, adr, write_tile, v_ref):
        h = j - first_step
        out, alpha = _gat_step(gin, w_ref[...], asr, adr, cs[...], h)
        write_tile(h, out)

        @pl.when(h == 0)
        def _():
            vacc[...] = alpha * (1.0 / H)

        @pl.when(h > 0)
        def _():
            vacc[...] += alpha * (1.0 / H)

        @pl.when(h == H - 1)
        def _():
            v_ref[...] = vacc[...]

    @pl.when((j >= 1) & (j <= 8))
    def _():
        layer(1, g0s[...], w0_ref, as0_ref, ad0_ref,
              lambda h, o: gas.__setitem__((slice(None), pl.ds(h * HID, HID)), o),
              v1_ref)

    @pl.when((j >= 9) & (j <= 16))
    def _():
        layer(9, gas[...], w1_ref, as1_ref, ad1_ref,
              lambda h, o: gbs.__setitem__((slice(None), pl.ds(h * HID, HID)), o),
              v2_ref)

    @pl.when(j >= 17)
    def _():
        layer(17, gbs[...], w2_ref, as2_ref, ad2_ref,
              lambda h, o: g3_ref.__setitem__((slice(None), pl.ds(h * HID, HID)), o),
              v3_ref)


def _k1(xp, pep, gat):
    const2 = lambda _: (0, 0)
    return pl.pallas_call(
        _k1_body,
        grid=(1 + 3 * H,),
        in_specs=[
            pl.BlockSpec((NP, IN_DIM), const2),
            pl.BlockSpec((NP, IN_DIM), const2),
            pl.BlockSpec((IN_DIM, HID), lambda j: (0, jnp.clip(j - 1, 0, H - 1))),
            pl.BlockSpec((D, HID), lambda j: (0, jnp.clip(j - 9, 0, H - 1))),
            pl.BlockSpec((D, HID), lambda j: (0, jnp.clip(j - 17, 0, H - 1))),
            pl.BlockSpec((H, HID), const2), pl.BlockSpec((H, HID), const2),
            pl.BlockSpec((H, HID), const2), pl.BlockSpec((H, HID), const2),
            pl.BlockSpec((H, HID), const2), pl.BlockSpec((H, HID), const2),
        ],
        out_specs=(
            pl.BlockSpec((NP, NP), const2),
            pl.BlockSpec((NP, D), const2),
            pl.BlockSpec((NP, NP), const2),
            pl.BlockSpec((NP, NP), const2),
            pl.BlockSpec((NP, NP), const2),
            pl.BlockSpec((8, 128), const2),
        ),
        out_shape=(
            jax.ShapeDtypeStruct((NP, NP), F32),
            jax.ShapeDtypeStruct((NP, D), F32),
            jax.ShapeDtypeStruct((NP, NP), F32),
            jax.ShapeDtypeStruct((NP, NP), F32),
            jax.ShapeDtypeStruct((NP, NP), F32),
            jax.ShapeDtypeStruct((8, 128), F32),
        ),
        scratch_shapes=[
            pltpu.VMEM((NP, IN_DIM), F32),
            pltpu.VMEM((NP, D), F32),
            pltpu.VMEM((NP, D), F32),
            pltpu.VMEM((NP, NP), F32),
            pltpu.VMEM((NP, NP), F32),
        ],
    )(xp, pep, gat[0]['W'], gat[1]['W'], gat[2]['W'],
      gat[0]['a_src'], gat[0]['a_dst'], gat[1]['a_src'], gat[1]['a_dst'],
      gat[2]['a_src'], gat[2]['a_dst'])


def _k2_body(g_ref, win_ref, wout_ref, x1_ref, qkvs, asc, accs):
    j = pl.program_id(0)

    @pl.when(j < 3 * H)
    def _():
        qkvs[:, pl.ds(j * HID, HID)] = _dot(g_ref[...], win_ref[...],
                                            _CONTRACT_11)

    @pl.when((j >= 3 * H) & (j < 4 * H))
    def _():
        h = j - 3 * H
        qh = qkvs[:, pl.ds(h * HID, HID)]
        kh = qkvs[:, pl.ds((h + H) * HID, HID)]
        vh = qkvs[:, pl.ds((h + 2 * H) * HID, HID)]
        logits = _dot(qh, kh, _CONTRACT_11) * (1.0 / jnp.sqrt(HID * 1.0))
        colmask = lax.broadcasted_iota(jnp.int32, (NP, NP), 1) < N
        logits = jnp.where(colmask, logits, NEG)
        m = jnp.max(logits, axis=1, keepdims=True)
        e = jnp.exp(logits - m)
        e = jnp.where(colmask, e, 0.0)
        att = e / jnp.sum(e, axis=1, keepdims=True)
        asc[:, pl.ds(h * HID, HID)] = _dot(att, vh, _CONTRACT_10)

    @pl.when(j >= 4 * H)
    def _():
        jj = j - 4 * H
        accs[:, pl.ds(jj * HID, HID)] = _dot(asc[...], wout_ref[...],
                                             _CONTRACT_11)

        @pl.when(jj == H - 1)
        def _():
            x1_ref[...] = _ln(g_ref[...] + accs[...])


def _k2(g3, w_in, w_out):
    const2 = lambda _: (0, 0)
    return pl.pallas_call(
        _k2_body,
        grid=(5 * H,),
        in_specs=[
            pl.BlockSpec((NP, D), const2),
            pl.BlockSpec((HID, D), lambda j: (jnp.clip(j, 0, 3 * H - 1), 0)),
            pl.BlockSpec((HID, D), lambda j: (jnp.clip(j - 4 * H, 0, H - 1), 0)),
        ],
        out_specs=pl.BlockSpec((NP, D), const2),
        out_shape=jax.ShapeDtypeStruct((NP, D), F32),
        scratch_shapes=[
            pltpu.VMEM((NP, 3 * D), F32),
            pltpu.VMEM((NP, D), F32),
            pltpu.VMEM((NP, D), F32),
        ],
    )(g3, w_in, w_out)


def _edge_entropy(v, cmat, mask):
    vm = jnp.where(mask, v, NEG)
    mx = jnp.max(vm)
    e = jnp.where(mask, jnp.exp(v - mx), 0.0)
    s = jnp.sum(cmat * e)
    pr = e / s
    term = jnp.where(mask, pr * jnp.log(pr + 1e-8), 0.0)
    return -jnp.sum(cmat * term)


_NSLOT = 4
_NBLK = 13


def _k3_body(x1_ref, w1_ref, w2_ref, wo_ref, c_ref, v1_ref, v2_ref, v3_ref,
             out_ref, st_ref, slots, fs, accs, sems):
    nf = FF // HID

    def copy(b):
        s = b % _NSLOT
        if b < nf:
            return pltpu.make_async_copy(
                w1_ref.at[pl.ds(b * HID, HID), :], slots.at[s], sems.at[s])
        if b < nf + H:
            jj = b - nf
            return pltpu.make_async_copy(
                w2_ref.at[pl.ds(jj * HID, HID), :],
                slots.at[s, :, pl.ds(0, FF)], sems.at[s])
        return pltpu.make_async_copy(
            wo_ref, slots.at[s, pl.ds(0, OUT_DIM), :], sems.at[s])

    for b in range(_NSLOT - 1):
        copy(b).start()

    for b in range(_NBLK):
        copy(b).wait()
        if b + _NSLOT - 1 < _NBLK:
            copy(b + _NSLOT - 1).start()
        s = b % _NSLOT
        if b < nf:
            r = _dot(x1_ref[...], slots[s], _CONTRACT_11)
            fs[:, pl.ds(b * HID, HID)] = jnp.maximum(r, 0.0)
        elif b < nf + H:
            jj = b - nf
            accs[:, pl.ds(jj * HID, HID)] = _dot(
                fs[...], slots[s][:, :FF], _CONTRACT_11)
        else:
            t = _ln(x1_ref[...] + accs[...])
            validr = lax.broadcasted_iota(jnp.int32, (NP, D), 0) < N
            tmean = jnp.sum(jnp.where(validr, t, 0.0), axis=0,
                            keepdims=True) / N
            out = _dot(tmean, slots[s][:OUT_DIM, :], _CONTRACT_11)
            out_ref[...] = jnp.broadcast_to(out, (8, OUT_DIM))

            cmat = c_ref[...]
            mask = cmat > 0.0
            ent = (_edge_entropy(v1_ref[...], cmat, mask)
                   + _edge_entropy(v2_ref[...], cmat, mask)
                   + _edge_entropy(v3_ref[...], cmat, mask)) / 3.0
            rw = lax.broadcasted_iota(jnp.int32, (8, 128), 0)
            lane = lax.broadcasted_iota(jnp.int32, (8, 128), 1)
            st_ref[...] = jnp.where((rw == 0) & (lane == 0), ent, 0.0)


def _k3(x1, w1, w2, wo, cmat, v1, v2, v3):
    vm = pl.BlockSpec(memory_space=pltpu.VMEM)
    anym = pl.BlockSpec(memory_space=pl.ANY)
    return pl.pallas_call(
        _k3_body,
        in_specs=[vm, anym, anym, anym, vm, vm, vm, vm],
        out_specs=(vm, vm),
        out_shape=(
            jax.ShapeDtypeStruct((8, OUT_DIM), F32),
            jax.ShapeDtypeStruct((8, 128), F32),
        ),
        scratch_shapes=[
            pltpu.VMEM((_NSLOT, HID, D), F32),
            pltpu.VMEM((NP, FF), F32),
            pltpu.VMEM((NP, D), F32),
            pltpu.SemaphoreType.DMA((_NSLOT,)),
        ],
    )(x1, w1, w2, wo, cmat, v1, v2, v3)


def kernel(x, params):
    xp = jnp.pad(x, ((0, NP - N), (0, 0)))
    pep = jnp.pad(params['topo_pe'][:N, :IN_DIM], ((0, NP - N), (0, 0)))

    cmat, g3, v1, v2, v3, stats = _k1(xp, pep, params['gat'])
    x1 = _k2(g3, params['in_proj_w'], params['out_proj_w'])
    outr, st2 = _k3(x1, params['ff1_w'], params['ff2_w'], params['outp_w'],
                    cmat, v1, v2, v3)

    out = outr[0]
    return (out, stats[0, 0], stats[0, 1], st2[0, 0], stats[0, 2], stats[0, 3])


# speedup vs baseline: 1.3505x; 1.0945x over previous
import jax
import jax.numpy as jnp
from jax import lax
from jax.experimental import pallas as pl
from jax.experimental.pallas import tpu as pltpu

N = 100
NP = 128
IN_DIM = 512
HID = 512
H = 8
D = HID * H
FF = HID * 4
OUT_DIM = 256
K = 10
NEG = -1e30
F32 = jnp.float32

_CONTRACT_11 = (((1,), (1,)), ((), ()))
_CONTRACT_10 = (((1,), (0,)), ((), ()))


def _dot(a, b, dims):
    return lax.dot_general(a, b, dims, preferred_element_type=F32)


def _ln(xa):
    mu = jnp.mean(xa, axis=1, keepdims=True)
    var = jnp.mean((xa - mu) ** 2, axis=1, keepdims=True)
    return (xa - mu) / jnp.sqrt(var + 1e-5)


def _graph_setup(x, c_ref, stats_ref, cs):
    sims = _dot(x, x, _CONTRACT_11)
    col = lax.broadcasted_iota(jnp.int32, (NP, NP), 1)
    row = lax.broadcasted_iota(jnp.int32, (NP, NP), 0)
    valid_col = col < N
    valid_row = row < N

    selected = jnp.zeros((NP, NP), jnp.bool_)
    t_mat = jnp.zeros((NP, NP), F32)
    for t in range(K + 1):
        masked = jnp.where(valid_col & (~selected), sims, NEG)
        rowmax = jnp.max(masked, axis=1, keepdims=True)
        cand = jnp.where(masked == rowmax, col, NP)
        first = jnp.min(cand, axis=1, keepdims=True)
        newsel = col == first
        selected = selected | newsel
        if t > 0:
            t_mat = t_mat + newsel.astype(F32)
    t_mat = jnp.where(valid_row, t_mat, 0.0)
    cmat = t_mat.T + jnp.where((row == col) & valid_row, 1.0, 0.0)
    c_ref[...] = cmat
    cs[...] = cmat

    centrality = jnp.sum(sims, axis=1, keepdims=True)
    validr1 = lax.broadcasted_iota(jnp.int32, (NP, 1), 0) < N
    cmean = jnp.sum(centrality) / N
    cvar = jnp.sum(jnp.where(validr1, (centrality - cmean) ** 2, 0.0)) / (N - 1)
    cstd = jnp.sqrt(cvar)
    smean = jnp.sum(sims) / (N * N)
    degree = jnp.sum((sims > 0.5).astype(F32), axis=1, keepdims=True)
    s2 = _dot(sims, sims, _CONTRACT_10)
    tri = jnp.sum(s2 * sims, axis=1, keepdims=True)
    clus = tri / (degree * (degree - 1.0) + 1e-8)
    clustering = jnp.sum(jnp.where(validr1, clus, 0.0)) / N

    lane = lax.broadcasted_iota(jnp.int32, (8, 128), 1)
    rw = lax.broadcasted_iota(jnp.int32, (8, 128), 0)
    stats_ref[...] = jnp.where((rw == 0) & (lane == 0), cmean,
                     jnp.where((rw == 0) & (lane == 1), clustering,
                     jnp.where((rw == 0) & (lane == 2), smean,
                     jnp.where((rw == 0) & (lane == 3), cstd, 0.0))))


def _gat_step(g, w, asrc_ref, adst_ref, cmat, h):
    xp = _dot(g, w, _CONTRACT_10)
    asr = asrc_ref[pl.ds(h, 1), :]
    adr = adst_ref[pl.ds(h, 1), :]
    a_s_row = _dot(asr, xp, _CONTRACT_11)
    a_d_col = _dot(xp, adr, _CONTRACT_11)
    e = a_s_row + a_d_col
    e = jnp.where(e >= 0, e, 0.2 * e)
    mask = cmat > 0.0
    em = jnp.where(mask, e, NEG)
    m = jnp.max(em, axis=1, keepdims=True)
    m = jnp.where(m > 0.5 * NEG, m, 0.0)
    ex = jnp.where(mask, jnp.exp(e - m), 0.0)
    z = jnp.sum(cmat * ex, axis=1, keepdims=True)
    alpha = ex / (z + 1e-16)
    out = _dot(cmat * alpha, xp, _CONTRACT_10)
    out = jnp.where(out > 0, out, jnp.exp(out) - 1.0)
    validr = lax.broadcasted_iota(jnp.int32, (NP, HID), 0) < N
    return jnp.where(validr, out, 0.0), alpha


def _k1_body(x_ref, pe_ref, w0_ref, w1_ref, w2_ref,
             as0_ref, ad0_ref, as1_ref, ad1_ref, as2_ref, ad2_ref,
             c_ref, g3_ref, v1_ref, v2_ref, v3_ref, stats_ref,
             g0s, gas, gbs, cs, vacc, slots, sems):
    def copy(b):
        s = b % _NSLOT
        if b < H:
            return pltpu.make_async_copy(
                w0_ref.at[:, pl.ds(b * HID, HID)],
                slots.at[s, pl.ds(0, IN_DIM), :], sems.at[s])
        if b < 2 * H:
            return pltpu.make_async_copy(
                w1_ref.at[:, pl.ds((b - H) * HID, HID)], slots.at[s],
                sems.at[s])
        return pltpu.make_async_copy(
            w2_ref.at[:, pl.ds((b - 2 * H) * HID, HID)], slots.at[s],
            sems.at[s])

    for b in range(_NSLOT - 1):
        copy(b).start()

    x = x_ref[...]
    _graph_setup(x, c_ref, stats_ref, cs)
    validg = lax.broadcasted_iota(jnp.int32, (NP, IN_DIM), 0) < N
    g0s[...] = jnp.where(validg, x + pe_ref[...], 0.0)

    def head_epilogue(h, alpha, v_ref):
        if h == 0:
            vacc[...] = alpha * (1.0 / H)
        else:
            vacc[...] += alpha * (1.0 / H)
        if h == H - 1:
            v_ref[...] = vacc[...]

    for b in range(3 * H):
        copy(b).wait()
        if b + _NSLOT - 1 < 3 * H:
            copy(b + _NSLOT - 1).start()
        s = b % _NSLOT
        h = b % H
        if b < H:
            out, alpha = _gat_step(g0s[...], slots[s][:IN_DIM, :],
                                   as0_ref, ad0_ref, cs[...], h)
            gas[:, pl.ds(h * HID, HID)] = out
            head_epilogue(h, alpha, v1_ref)
        elif b < 2 * H:
            out, alpha = _gat_step(gas[...], slots[s],
                                   as1_ref, ad1_ref, cs[...], h)
            gbs[:, pl.ds(h * HID, HID)] = out
            head_epilogue(h, alpha, v2_ref)
        else:
            out, alpha = _gat_step(gbs[...], slots[s],
                                   as2_ref, ad2_ref, cs[...], h)
            g3_ref[:, pl.ds(h * HID, HID)] = out
            head_epilogue(h, alpha, v3_ref)


def _k1(xp, pep, gat):
    vm = pl.BlockSpec(memory_space=pltpu.VMEM)
    anym = pl.BlockSpec(memory_space=pl.ANY)
    return pl.pallas_call(
        _k1_body,
        in_specs=[vm, vm, anym, anym, anym, vm, vm, vm, vm, vm, vm],
        out_specs=(vm, vm, vm, vm, vm, vm),
        out_shape=(
            jax.ShapeDtypeStruct((NP, NP), F32),
            jax.ShapeDtypeStruct((NP, D), F32),
            jax.ShapeDtypeStruct((NP, NP), F32),
            jax.ShapeDtypeStruct((NP, NP), F32),
            jax.ShapeDtypeStruct((NP, NP), F32),
            jax.ShapeDtypeStruct((8, 128), F32),
        ),
        scratch_shapes=[
            pltpu.VMEM((NP, IN_DIM), F32),
            pltpu.VMEM((NP, D), F32),
            pltpu.VMEM((NP, D), F32),
            pltpu.VMEM((NP, NP), F32),
            pltpu.VMEM((NP, NP), F32),
            pltpu.VMEM((_NSLOT, D, HID), F32),
            pltpu.SemaphoreType.DMA((_NSLOT,)),
        ],
    )(xp, pep, gat[0]['W'], gat[1]['W'], gat[2]['W'],
      gat[0]['a_src'], gat[0]['a_dst'], gat[1]['a_src'], gat[1]['a_dst'],
      gat[2]['a_src'], gat[2]['a_dst'])


def _attn_head(qkvs, asc, h):
    qh = qkvs[:, pl.ds(h * HID, HID)]
    kh = qkvs[:, pl.ds((h + H) * HID, HID)]
    vh = qkvs[:, pl.ds((h + 2 * H) * HID, HID)]
    logits = _dot(qh, kh, _CONTRACT_11) * (1.0 / jnp.sqrt(HID * 1.0))
    colmask = lax.broadcasted_iota(jnp.int32, (NP, NP), 1) < N
    logits = jnp.where(colmask, logits, NEG)
    m = jnp.max(logits, axis=1, keepdims=True)
    e = jnp.exp(logits - m)
    e = jnp.where(colmask, e, 0.0)
    att = e / jnp.sum(e, axis=1, keepdims=True)
    asc[:, pl.ds(h * HID, HID)] = _dot(att, vh, _CONTRACT_10)


def _k2_body(g_ref, win_ref, wout_ref, x1_ref, qkvs, asc, accs, slots, sems):
    def copy(b):
        s = b % _NSLOT
        if b < 3 * H:
            src = win_ref.at[pl.ds(b * HID, HID), :]
        else:
            src = wout_ref.at[pl.ds((b - 3 * H) * HID, HID), :]
        return pltpu.make_async_copy(src, slots.at[s], sems.at[s])

    for b in range(_NSLOT - 1):
        copy(b).start()

    for b in range(4 * H):
        copy(b).wait()
        if b + _NSLOT - 1 < 4 * H:
            copy(b + _NSLOT - 1).start()
        s = b % _NSLOT
        if b < 3 * H:
            qkvs[:, pl.ds(b * HID, HID)] = _dot(g_ref[...], slots[s],
                                                _CONTRACT_11)
            if b >= 2 * H + 1:
                _attn_head(qkvs, asc, b - (2 * H + 1))
        else:
            jj = b - 3 * H
            if jj == 0:
                _attn_head(qkvs, asc, H - 1)
            accs[:, pl.ds(jj * HID, HID)] = _dot(asc[...], slots[s],
                                                 _CONTRACT_11)
            if jj == H - 1:
                x1_ref[...] = _ln(g_ref[...] + accs[...])


def _k2(g3, w_in, w_out):
    vm = pl.BlockSpec(memory_space=pltpu.VMEM)
    anym = pl.BlockSpec(memory_space=pl.ANY)
    return pl.pallas_call(
        _k2_body,
        in_specs=[vm, anym, anym],
        out_specs=vm,
        out_shape=jax.ShapeDtypeStruct((NP, D), F32),
        scratch_shapes=[
            pltpu.VMEM((NP, 3 * D), F32),
            pltpu.VMEM((NP, D), F32),
            pltpu.VMEM((NP, D), F32),
            pltpu.VMEM((_NSLOT, HID, D), F32),
            pltpu.SemaphoreType.DMA((_NSLOT,)),
        ],
    )(g3, w_in, w_out)


def _edge_entropy(v, cmat, mask):
    vm = jnp.where(mask, v, NEG)
    mx = jnp.max(vm)
    e = jnp.where(mask, jnp.exp(v - mx), 0.0)
    s = jnp.sum(cmat * e)
    pr = e / s
    term = jnp.where(mask, pr * jnp.log(pr + 1e-8), 0.0)
    return -jnp.sum(cmat * term)


_NSLOT = 4
_NBLK = 13


def _k3_body(x1_ref, w1_ref, w2_ref, wo_ref, c_ref, v1_ref, v2_ref, v3_ref,
             out_ref, st_ref, slots, fs, accs, sems):
    nf = FF // HID

    def copy(b):
        s = b % _NSLOT
        if b < nf:
            return pltpu.make_async_copy(
                w1_ref.at[pl.ds(b * HID, HID), :], slots.at[s], sems.at[s])
        if b < nf + H:
            jj = b - nf
            return pltpu.make_async_copy(
                w2_ref.at[pl.ds(jj * HID, HID), :],
                slots.at[s, :, pl.ds(0, FF)], sems.at[s])
        return pltpu.make_async_copy(
            wo_ref, slots.at[s, pl.ds(0, OUT_DIM), :], sems.at[s])

    for b in range(_NSLOT - 1):
        copy(b).start()

    for b in range(_NBLK):
        copy(b).wait()
        if b + _NSLOT - 1 < _NBLK:
            copy(b + _NSLOT - 1).start()
        s = b % _NSLOT
        if b < nf:
            r = _dot(x1_ref[...], slots[s], _CONTRACT_11)
            fs[:, pl.ds(b * HID, HID)] = jnp.maximum(r, 0.0)
        elif b < nf + H:
            jj = b - nf
            accs[:, pl.ds(jj * HID, HID)] = _dot(
                fs[...], slots[s][:, :FF], _CONTRACT_11)
        else:
            t = _ln(x1_ref[...] + accs[...])
            validr = lax.broadcasted_iota(jnp.int32, (NP, D), 0) < N
            tmean = jnp.sum(jnp.where(validr, t, 0.0), axis=0,
                            keepdims=True) / N
            out = _dot(tmean, slots[s][:OUT_DIM, :], _CONTRACT_11)
            out_ref[...] = jnp.broadcast_to(out, (8, OUT_DIM))

            cmat = c_ref[...]
            mask = cmat > 0.0
            ent = (_edge_entropy(v1_ref[...], cmat, mask)
                   + _edge_entropy(v2_ref[...], cmat, mask)
                   + _edge_entropy(v3_ref[...], cmat, mask)) / 3.0
            rw = lax.broadcasted_iota(jnp.int32, (8, 128), 0)
            lane = lax.broadcasted_iota(jnp.int32, (8, 128), 1)
            st_ref[...] = jnp.where((rw == 0) & (lane == 0), ent, 0.0)


def _k3(x1, w1, w2, wo, cmat, v1, v2, v3):
    vm = pl.BlockSpec(memory_space=pltpu.VMEM)
    anym = pl.BlockSpec(memory_space=pl.ANY)
    return pl.pallas_call(
        _k3_body,
        in_specs=[vm, anym, anym, anym, vm, vm, vm, vm],
        out_specs=(vm, vm),
        out_shape=(
            jax.ShapeDtypeStruct((8, OUT_DIM), F32),
            jax.ShapeDtypeStruct((8, 128), F32),
        ),
        scratch_shapes=[
            pltpu.VMEM((_NSLOT, HID, D), F32),
            pltpu.VMEM((NP, FF), F32),
            pltpu.VMEM((NP, D), F32),
            pltpu.SemaphoreType.DMA((_NSLOT,)),
        ],
    )(x1, w1, w2, wo, cmat, v1, v2, v3)


def kernel(x, params):
    xp = jnp.pad(x, ((0, NP - N), (0, 0)))
    pep = jnp.pad(params['topo_pe'][:N, :IN_DIM], ((0, NP - N), (0, 0)))

    cmat, g3, v1, v2, v3, stats = _k1(xp, pep, params['gat'])
    x1 = _k2(g3, params['in_proj_w'], params['out_proj_w'])
    outr, st2 = _k3(x1, params['ff1_w'], params['ff2_w'], params['outp_w'],
                    cmat, v1, v2, v3)

    out = outr[0]
    return (out, stats[0, 0], stats[0, 1], st2[0, 0], stats[0, 2], stats[0, 3])


# speedup vs baseline: 1.5143x; 1.1212x over previous
import jax
import jax.numpy as jnp
from jax import lax
from jax.experimental import pallas as pl
from jax.experimental.pallas import tpu as pltpu

N = 100
NP = 128
IN_DIM = 512
HID = 512
H = 8
D = HID * H
FF = HID * 4
OUT_DIM = 256
K = 10
NEG = -1e30
F32 = jnp.float32

_CONTRACT_11 = (((1,), (1,)), ((), ()))
_CONTRACT_10 = (((1,), (0,)), ((), ()))

_NSLOT = 4
_B_W1, _B_W2, _B_QKV, _B_PROJ, _B_FF1, _B_FF2, _B_OUTP = 1, 9, 17, 41, 49, 53, 61
_NBLK = 62


def _dot(a, b, dims):
    return lax.dot_general(a, b, dims, preferred_element_type=F32)


def _ln(xa):
    mu = jnp.mean(xa, axis=1, keepdims=True)
    var = jnp.mean((xa - mu) ** 2, axis=1, keepdims=True)
    return (xa - mu) / jnp.sqrt(var + 1e-5)


def _graph_setup(x, stats_ref, cs):
    sims = _dot(x, x, _CONTRACT_11)
    col = lax.broadcasted_iota(jnp.int32, (NP, NP), 1)
    row = lax.broadcasted_iota(jnp.int32, (NP, NP), 0)
    valid_col = col < N
    valid_row = row < N

    selected = jnp.zeros((NP, NP), jnp.bool_)
    t_mat = jnp.zeros((NP, NP), F32)
    for t in range(K + 1):
        masked = jnp.where(valid_col & (~selected), sims, NEG)
        rowmax = jnp.max(masked, axis=1, keepdims=True)
        cand = jnp.where(masked == rowmax, col, NP)
        first = jnp.min(cand, axis=1, keepdims=True)
        newsel = col == first
        selected = selected | newsel
        if t > 0:
            t_mat = t_mat + newsel.astype(F32)
    t_mat = jnp.where(valid_row, t_mat, 0.0)
    cs[...] = t_mat.T + jnp.where((row == col) & valid_row, 1.0, 0.0)

    centrality = jnp.sum(sims, axis=1, keepdims=True)
    validr1 = lax.broadcasted_iota(jnp.int32, (NP, 1), 0) < N
    cmean = jnp.sum(centrality) / N
    cvar = jnp.sum(jnp.where(validr1, (centrality - cmean) ** 2, 0.0)) / (N - 1)
    cstd = jnp.sqrt(cvar)
    smean = jnp.sum(sims) / (N * N)
    degree = jnp.sum((sims > 0.5).astype(F32), axis=1, keepdims=True)
    s2 = _dot(sims, sims, _CONTRACT_10)
    tri = jnp.sum(s2 * sims, axis=1, keepdims=True)
    clus = tri / (degree * (degree - 1.0) + 1e-8)
    clustering = jnp.sum(jnp.where(validr1, clus, 0.0)) / N

    lane = lax.broadcasted_iota(jnp.int32, (8, 128), 1)
    rw = lax.broadcasted_iota(jnp.int32, (8, 128), 0)
    stats_ref[...] = jnp.where((rw == 0) & (lane == 0), cmean,
                     jnp.where((rw == 0) & (lane == 1), clustering,
                     jnp.where((rw == 0) & (lane == 2), smean,
                     jnp.where((rw == 0) & (lane == 3), cstd, 0.0))))


def _gat_attend(xp, asrc_ref, adst_ref, cmat, h):
    asr = asrc_ref[pl.ds(h, 1), :]
    adr = adst_ref[pl.ds(h, 1), :]
    a_s_row = _dot(asr, xp, _CONTRACT_11)
    a_d_col = _dot(xp, adr, _CONTRACT_11)
    e = a_s_row + a_d_col
    e = jnp.where(e >= 0, e, 0.2 * e)
    mask = cmat > 0.0
    em = jnp.where(mask, e, NEG)
    m = jnp.max(em, axis=1, keepdims=True)
    m = jnp.where(m > 0.5 * NEG, m, 0.0)
    ex = jnp.where(mask, jnp.exp(e - m), 0.0)
    z = jnp.sum(cmat * ex, axis=1, keepdims=True)
    alpha = ex / (z + 1e-16)
    out = _dot(cmat * alpha, xp, _CONTRACT_10)
    out = jnp.where(out > 0, out, jnp.exp(out) - 1.0)
    validr = lax.broadcasted_iota(jnp.int32, (NP, HID), 0) < N
    return jnp.where(validr, out, 0.0), alpha


def _attn_head(qkvs, asc, h):
    qh = qkvs[:, pl.ds(h * HID, HID)]
    kh = qkvs[:, pl.ds((h + H) * HID, HID)]
    vh = qkvs[:, pl.ds((h + 2 * H) * HID, HID)]
    logits = _dot(qh, kh, _CONTRACT_11) * (1.0 / jnp.sqrt(HID * 1.0))
    colmask = lax.broadcasted_iota(jnp.int32, (NP, NP), 1) < N
    logits = jnp.where(colmask, logits, NEG)
    m = jnp.max(logits, axis=1, keepdims=True)
    e = jnp.exp(logits - m)
    e = jnp.where(colmask, e, 0.0)
    att = e / jnp.sum(e, axis=1, keepdims=True)
    asc[:, pl.ds(h * HID, HID)] = _dot(att, vh, _CONTRACT_10)


def _edge_entropy(v, cmat, mask):
    vm = jnp.where(mask, v, NEG)
    mx = jnp.max(vm)
    e = jnp.where(mask, jnp.exp(v - mx), 0.0)
    s = jnp.sum(cmat * e)
    pr = e / s
    term = jnp.where(mask, pr * jnp.log(pr + 1e-8), 0.0)
    return -jnp.sum(cmat * term)


def _mega_body(x_ref, pe_ref, w0_ref, w1_ref, w2_ref,
               as0_ref, ad0_ref, as1_ref, ad1_ref, as2_ref, ad2_ref,
               win_ref, wout_ref, wf1_ref, wf2_ref, wo_ref,
               out_ref, stats_ref,
               g0s, gas, gbs, xpa, qkvs, asc, accs, x1s, fs, cs, vs,
               slots, sems):
    def copy(b):
        s = b % _NSLOT
        if b == 0:
            return pltpu.make_async_copy(w0_ref, slots.at[s], sems.at[s])
        if b < _B_W2:
            src = w1_ref.at[pl.ds((b - _B_W1) * HID, HID), :]
        elif b < _B_QKV:
            src = w2_ref.at[pl.ds((b - _B_W2) * HID, HID), :]
        elif b < _B_PROJ:
            src = win_ref.at[pl.ds((b - _B_QKV) * HID, HID), :]
        elif b < _B_FF1:
            src = wout_ref.at[pl.ds((b - _B_PROJ) * HID, HID), :]
        elif b < _B_FF2:
            src = wf1_ref.at[pl.ds((b - _B_FF1) * HID, HID), :]
        elif b < _B_OUTP:
            return pltpu.make_async_copy(
                wf2_ref.at[pl.ds((b - _B_FF2) * HID, HID), :],
                slots.at[s, :, pl.ds(0, FF)], sems.at[s])
        else:
            return pltpu.make_async_copy(
                wo_ref, slots.at[s, pl.ds(0, OUT_DIM), :], sems.at[s])
        return pltpu.make_async_copy(src, slots.at[s], sems.at[s])

    for b in range(_NSLOT - 1):
        copy(b).start()

    x = x_ref[...]
    _graph_setup(x, stats_ref, cs)
    validg = lax.broadcasted_iota(jnp.int32, (NP, IN_DIM), 0) < N
    g0s[...] = jnp.where(validg, x + pe_ref[...], 0.0)

    def gat_attn_all(asrc_ref, adst_ref, gout, lidx):
        vsum = None
        for h in range(H):
            xph = xpa[:, pl.ds(h * HID, HID)]
            out, alpha = _gat_attend(xph, asrc_ref, adst_ref, cs[...], h)
            gout[:, pl.ds(h * HID, HID)] = out
            vsum = alpha * (1.0 / H) if vsum is None else vsum + alpha * (1.0 / H)
        vs[lidx] = vsum

    for b in range(_NBLK):
        copy(b).wait()
        if b + _NSLOT - 1 < _NBLK:
            copy(b + _NSLOT - 1).start()
        s = b % _NSLOT
        w = slots[s]
        if b == 0:
            xpa[...] = _dot(g0s[...], w, _CONTRACT_10)
            gat_attn_all(as0_ref, ad0_ref, gas, 0)
        elif b < _B_W2:
            r = b - _B_W1
            part = _dot(gas[:, pl.ds(r * HID, HID)], w, _CONTRACT_10)
            if r == 0:
                xpa[...] = part
            else:
                xpa[...] += part
            if r == H - 1:
                gat_attn_all(as1_ref, ad1_ref, gbs, 1)
        elif b < _B_QKV:
            r = b - _B_W2
            part = _dot(gbs[:, pl.ds(r * HID, HID)], w, _CONTRACT_10)
            if r == 0:
                xpa[...] = part
            else:
                xpa[...] += part
            if r == H - 1:
                gat_attn_all(as2_ref, ad2_ref, gas, 2)
        elif b < _B_PROJ:
            r = b - _B_QKV
            qkvs[:, pl.ds(r * HID, HID)] = _dot(gas[...], w, _CONTRACT_11)
            if r >= 2 * H + 1:
                _attn_head(qkvs, asc, r - (2 * H + 1))
        elif b < _B_FF1:
            jj = b - _B_PROJ
            if jj == 0:
                _attn_head(qkvs, asc, H - 1)
            accs[:, pl.ds(jj * HID, HID)] = _dot(asc[...], w, _CONTRACT_11)
            if jj == H - 1:
                x1s[...] = _ln(gas[...] + accs[...])
        elif b < _B_FF2:
            r = b - _B_FF1
            part = _dot(x1s[...], w, _CONTRACT_11)
            fs[:, pl.ds(r * HID, HID)] = jnp.maximum(part, 0.0)
        elif b < _B_OUTP:
            jj = b - _B_FF2
            accs[:, pl.ds(jj * HID, HID)] = _dot(fs[...], slots[s][:, :FF],
                                                 _CONTRACT_11)
        else:
            t = _ln(x1s[...] + accs[...])
            validr = lax.broadcasted_iota(jnp.int32, (NP, D), 0) < N
            tmean = jnp.sum(jnp.where(validr, t, 0.0), axis=0,
                            keepdims=True) / N
            out = _dot(tmean, slots[s][:OUT_DIM, :], _CONTRACT_11)
            out_ref[...] = jnp.broadcast_to(out, (8, OUT_DIM))

            cmat = cs[...]
            mask = cmat > 0.0
            ent = (_edge_entropy(vs[0], cmat, mask)
                   + _edge_entropy(vs[1], cmat, mask)
                   + _edge_entropy(vs[2], cmat, mask)) / 3.0
            lane = lax.broadcasted_iota(jnp.int32, (8, 128), 1)
            rw = lax.broadcasted_iota(jnp.int32, (8, 128), 0)
            stats_ref[...] = stats_ref[...] + jnp.where(
                (rw == 0) & (lane == 4), ent, 0.0)


def kernel(x, params):
    xp = jnp.pad(x, ((0, NP - N), (0, 0)))
    pep = jnp.pad(params['topo_pe'][:N, :IN_DIM], ((0, NP - N), (0, 0)))
    gat = params['gat']

    vm = pl.BlockSpec(memory_space=pltpu.VMEM)
    anym = pl.BlockSpec(memory_space=pl.ANY)
    outr, stats = pl.pallas_call(
        _mega_body,
        in_specs=[vm, vm, anym, anym, anym, vm, vm, vm, vm, vm, vm,
                  anym, anym, anym, anym, anym],
        out_specs=(vm, vm),
        out_shape=(
            jax.ShapeDtypeStruct((8, OUT_DIM), F32),
            jax.ShapeDtypeStruct((8, 128), F32),
        ),
        scratch_shapes=[
            pltpu.VMEM((NP, IN_DIM), F32),
            pltpu.VMEM((NP, D), F32),
            pltpu.VMEM((NP, D), F32),
            pltpu.VMEM((NP, D), F32),
            pltpu.VMEM((NP, 3 * D), F32),
            pltpu.VMEM((NP, D), F32),
            pltpu.VMEM((NP, D), F32),
            pltpu.VMEM((NP, D), F32),
            pltpu.VMEM((NP, FF), F32),
            pltpu.VMEM((NP, NP), F32),
            pltpu.VMEM((3, NP, NP), F32),
            pltpu.VMEM((_NSLOT, HID, D), F32),
            pltpu.SemaphoreType.DMA((_NSLOT,)),
        ],
    )(xp, pep, gat[0]['W'], gat[1]['W'], gat[2]['W'],
      gat[0]['a_src'], gat[0]['a_dst'], gat[1]['a_src'], gat[1]['a_dst'],
      gat[2]['a_src'], gat[2]['a_dst'],
      params['in_proj_w'], params['out_proj_w'],
      params['ff1_w'], params['ff2_w'], params['outp_w'])

    out = outr[0]
    return (out, stats[0, 0], stats[0, 1], stats[0, 4],
            stats[0, 2], stats[0, 3])
